# baseline (speedup 1.0000x reference)
import functools

import jax
import jax.numpy as jnp
from jax import lax
from jax.experimental import pallas as pl
from jax.experimental.pallas import tpu as pltpu
from jax.experimental.pallas import tpu_sc as plsc

SEQ = 8192
BATCH = 4
D = 768

NUM_CORES = 2
NUM_SUBCORES = 16
NW = NUM_CORES * NUM_SUBCORES
ROWS_PER_W = SEQ // NW
CHUNK = 64
NCHUNK = ROWS_PER_W // CHUNK


def _body(pe_hbm, out_hbm, buf0, buf1, sem0, sem1):
    wid = lax.axis_index("s") * NUM_CORES + lax.axis_index("c")
    base = wid * ROWS_PER_W
    bufs = (buf0, buf1)
    sems = (sem0, sem1)
    copies = [None, None]
    copies[0] = pltpu.async_copy(pe_hbm.at[pl.ds(base, CHUNK)], buf0, sem0)
    for i in range(NCHUNK):
        if i + 1 < NCHUNK:
            j = (i + 1) % 2
            copies[j] = pltpu.async_copy(
                pe_hbm.at[pl.ds(base + (i + 1) * CHUNK, CHUNK)], bufs[j], sems[j])
        copies[i % 2].wait()
        row0 = base + i * CHUNK
        cur = bufs[i % 2]
        for b in range(BATCH):
            pltpu.sync_copy(cur, out_hbm.at[pl.ds(row0, CHUNK), pl.ds(b * D, D)])


_bcast = functools.partial(
    pl.kernel,
    out_type=jax.ShapeDtypeStruct((SEQ, BATCH * D), jnp.float32),
    mesh=plsc.VectorSubcoreMesh(
        core_axis_name="c", subcore_axis_name="s",
        num_cores=NUM_CORES, num_subcores=NUM_SUBCORES),
    scratch_types=[
        pltpu.VMEM((CHUNK, D), jnp.float32),
        pltpu.VMEM((CHUNK, D), jnp.float32),
        pltpu.SemaphoreType.DMA,
        pltpu.SemaphoreType.DMA,
    ],
)(_body)


@jax.jit
def kernel(z, pos_embed):
    del z
    out2 = _bcast(pos_embed)
    return out2.reshape(SEQ, BATCH, D)


# speedup vs baseline: 2.8073x; 2.8073x over previous
import functools

import jax
import jax.numpy as jnp
from jax import lax
from jax.experimental import pallas as pl
from jax.experimental.pallas import tpu as pltpu
from jax.experimental.pallas import tpu_sc as plsc

SEQ = 8192
BATCH = 4
D = 768

NUM_CORES = 2
NUM_SUBCORES = 16
NW = NUM_CORES * NUM_SUBCORES
ROWS_PER_W = SEQ // NW
CHUNK = 64
NCHUNK = ROWS_PER_W // CHUNK


def _body(pe_hbm, out_hbm, buf0, buf1, sem0, sem1):
    wid = lax.axis_index("s") * NUM_CORES + lax.axis_index("c")
    base = wid * ROWS_PER_W
    bufs = (buf0, buf1)
    sems = (sem0, sem1)
    copies = [None, None]
    copies[0] = pltpu.async_copy(pe_hbm.at[pl.ds(base, CHUNK)], buf0, sem0)
    for i in range(NCHUNK):
        if i + 1 < NCHUNK:
            j = (i + 1) % 2
            copies[j] = pltpu.async_copy(
                pe_hbm.at[pl.ds(base + (i + 1) * CHUNK, CHUNK)], bufs[j], sems[j])
        copies[i % 2].wait()
        row0 = base + i * CHUNK
        cur = bufs[i % 2]
        for b in range(BATCH):
            pltpu.sync_copy(cur, out_hbm.at[pl.ds(row0, CHUNK), b])


_bcast = functools.partial(
    pl.kernel,
    out_type=jax.ShapeDtypeStruct((SEQ, BATCH, D), jnp.float32),
    mesh=plsc.VectorSubcoreMesh(
        core_axis_name="c", subcore_axis_name="s",
        num_cores=NUM_CORES, num_subcores=NUM_SUBCORES),
    scratch_types=[
        pltpu.VMEM((CHUNK, D), jnp.float32),
        pltpu.VMEM((CHUNK, D), jnp.float32),
        pltpu.SemaphoreType.DMA,
        pltpu.SemaphoreType.DMA,
    ],
)(_body)


@jax.jit
def kernel(z, pos_embed):
    del z
    return _bcast(pos_embed)
